# baseline (speedup 1.0000x reference)
import jax
import jax.numpy as jnp
from jax import lax
from jax.experimental import pallas as pl
from jax.experimental.pallas import tpu as pltpu

_B, _P, _D = 64, 576, 384
_CB = 4
_NC = _B // _CB
_RING = 8
_NR = _NC // _RING


def _body(in_hbm, tbl_hbm, out_hbm, tbl, in_bufs, out_bufs,
          tsem, in_sems, out_sems):
    pltpu.async_copy(tbl_hbm, tbl, tsem).wait()

    for k in range(_RING):
        pltpu.async_copy(in_hbm.at[pl.ds(k * _CB, _CB)], in_bufs.at[k],
                         in_sems.at[k])

    def round_body(r, carry):
        for k in range(_RING):
            c = r * _RING + k
            b = c * _CB
            pltpu.make_async_copy(in_hbm.at[pl.ds(b, _CB)], in_bufs.at[k],
                                  in_sems.at[k]).wait()

            @pl.when(r > 0)
            def _():
                pltpu.make_async_copy(out_bufs.at[k],
                                      out_hbm.at[pl.ds(b - _RING * _CB, _CB)],
                                      out_sems.at[k]).wait()

            out_bufs[k] = in_bufs[k] + tbl[...][None]
            pltpu.async_copy(out_bufs.at[k], out_hbm.at[pl.ds(b, _CB)],
                             out_sems.at[k])

            @pl.when(c + _RING < _NC)
            def _():
                pltpu.async_copy(in_hbm.at[pl.ds(b + _RING * _CB, _CB)],
                                 in_bufs.at[k], in_sems.at[k])

        return carry

    lax.fori_loop(0, _NR, round_body, 0)

    for k in range(_RING):
        b = (_NC - _RING + k) * _CB
        pltpu.make_async_copy(out_bufs.at[k], out_hbm.at[pl.ds(b, _CB)],
                              out_sems.at[k]).wait()


def kernel(inputs, table):
    B, P, D = inputs.shape
    return pl.pallas_call(
        _body,
        in_specs=[
            pl.BlockSpec(memory_space=pl.ANY),
            pl.BlockSpec(memory_space=pl.ANY),
        ],
        out_specs=pl.BlockSpec(memory_space=pl.ANY),
        out_shape=jax.ShapeDtypeStruct((B, P, D), inputs.dtype),
        scratch_shapes=[
            pltpu.VMEM((P, D), jnp.float32),
            pltpu.VMEM((_RING, _CB, P, D), jnp.float32),
            pltpu.VMEM((_RING, _CB, P, D), jnp.float32),
            pltpu.SemaphoreType.DMA,
            pltpu.SemaphoreType.DMA((_RING,)),
            pltpu.SemaphoreType.DMA((_RING,)),
        ],
    )(inputs, table)


# speedup vs baseline: 1.0097x; 1.0097x over previous
import jax
import jax.numpy as jnp
from jax import lax
from jax.experimental import pallas as pl
from jax.experimental.pallas import tpu as pltpu

_B, _P, _D = 64, 576, 384
_CB = 8
_NC = _B // _CB
_RING = 4
_NR = _NC // _RING


def _body(in_hbm, tbl_hbm, out_hbm, tbl, in_bufs, out_bufs,
          tsem, in_sems, out_sems):
    pltpu.async_copy(tbl_hbm, tbl, tsem).wait()

    for k in range(_RING):
        pltpu.async_copy(in_hbm.at[pl.ds(k * _CB, _CB)], in_bufs.at[k],
                         in_sems.at[k])

    def round_body(r, carry):
        for k in range(_RING):
            c = r * _RING + k
            b = c * _CB
            pltpu.make_async_copy(in_hbm.at[pl.ds(b, _CB)], in_bufs.at[k],
                                  in_sems.at[k]).wait()

            @pl.when(r > 0)
            def _():
                pltpu.make_async_copy(out_bufs.at[k],
                                      out_hbm.at[pl.ds(b - _RING * _CB, _CB)],
                                      out_sems.at[k]).wait()

            out_bufs[k] = in_bufs[k] + tbl[...][None]
            pltpu.async_copy(out_bufs.at[k], out_hbm.at[pl.ds(b, _CB)],
                             out_sems.at[k])

            @pl.when(c + _RING < _NC)
            def _():
                pltpu.async_copy(in_hbm.at[pl.ds(b + _RING * _CB, _CB)],
                                 in_bufs.at[k], in_sems.at[k])

        return carry

    lax.fori_loop(0, _NR, round_body, 0)

    for k in range(_RING):
        b = (_NC - _RING + k) * _CB
        pltpu.make_async_copy(out_bufs.at[k], out_hbm.at[pl.ds(b, _CB)],
                              out_sems.at[k]).wait()


def kernel(inputs, table):
    B, P, D = inputs.shape
    return pl.pallas_call(
        _body,
        in_specs=[
            pl.BlockSpec(memory_space=pl.ANY),
            pl.BlockSpec(memory_space=pl.ANY),
        ],
        out_specs=pl.BlockSpec(memory_space=pl.ANY),
        out_shape=jax.ShapeDtypeStruct((B, P, D), inputs.dtype),
        scratch_shapes=[
            pltpu.VMEM((P, D), jnp.float32),
            pltpu.VMEM((_RING, _CB, P, D), jnp.float32),
            pltpu.VMEM((_RING, _CB, P, D), jnp.float32),
            pltpu.SemaphoreType.DMA,
            pltpu.SemaphoreType.DMA((_RING,)),
            pltpu.SemaphoreType.DMA((_RING,)),
        ],
    )(inputs, table)


# speedup vs baseline: 1.0483x; 1.0382x over previous
import jax
import jax.numpy as jnp
from jax.experimental import pallas as pl


def _add_body(in_ref, table_ref, out_ref):
    out_ref[...] = in_ref[...] + table_ref[...][None]


def kernel(inputs, table):
    B, P, D = inputs.shape
    BB = 16
    return pl.pallas_call(
        _add_body,
        grid=(B // BB,),
        in_specs=[
            pl.BlockSpec((BB, P, D), lambda b: (b, 0, 0)),
            pl.BlockSpec((P, D), lambda b: (0, 0)),
        ],
        out_specs=pl.BlockSpec((BB, P, D), lambda b: (b, 0, 0)),
        out_shape=jax.ShapeDtypeStruct((B, P, D), inputs.dtype),
    )(inputs, table)
